# Initial kernel scaffold; baseline (speedup 1.0000x reference)
import jax
import jax.numpy as jnp
from jax.experimental import pallas as pl


def kernel(x_s, edge_index_s, edge_attr_s, batch_s, x_t, edge_index_t, edge_attr_t, batch_t, W, We, b):
    raise NotImplementedError("write your pallas kernel here")


# speedup vs baseline: 2.4203x; 2.4203x over previous
import functools

import jax
import jax.numpy as jnp
from jax import lax
from jax.experimental import pallas as pl
from jax.experimental.pallas import tpu as pltpu
from jax.experimental.pallas import tpu_sc as plsc

N = 10000
B = 8
NS = N // B
E = 320000
D = 128
DE = 16
K = 32

NPAD = 10240
NSP = 1280
EP = 327680
NHALF = NPAD // 2
NALLOC = 5248


def _matmul_body(x_ref, w_ref, o_ref):
    o_ref[...] = jnp.dot(x_ref[...], w_ref[...],
                         preferred_element_type=jnp.float32)


def _matmul(x, w, bm=1000):
    m, k = x.shape
    return pl.pallas_call(
        _matmul_body,
        grid=(m // bm,),
        in_specs=[pl.BlockSpec((bm, k), lambda i: (i, 0)),
                  pl.BlockSpec((k, w.shape[1]), lambda i: (0, 0))],
        out_specs=pl.BlockSpec((bm, w.shape[1]), lambda i: (i, 0)),
        out_shape=jax.ShapeDtypeStruct((m, w.shape[1]), jnp.float32),
    )(x, w)


def _make_segsum():
    mesh = plsc.VectorSubcoreMesh(core_axis_name="c", subcore_axis_name="s")

    @functools.partial(
        pl.kernel,
        mesh=mesh,
        out_type=jax.ShapeDtypeStruct((2 * NPAD, D), jnp.float32),
        scratch_types=[
            pltpu.VMEM_SHARED((NALLOC, D), jnp.float32),
            pltpu.VMEM((128,), jnp.int32),
            pltpu.VMEM((128,), jnp.int32),
            pltpu.VMEM((128,), jnp.int32),
            pltpu.VMEM((128, D), jnp.float32),
            pltpu.VMEM((128, D), jnp.float32),
            pltpu.SemaphoreType.DMA,
        ],
    )
    def seg(hpre, src2d, dst2d, eawe, zh, outh,
            aggh, srcb, dstb, dstl, rowsb, eweb, sem):
        c = lax.axis_index("c")
        s = lax.axis_index("s")
        lo = c * NHALF

        def graph_pass(g, carry):
            pltpu.sync_copy(zh.at[pl.ds(0, 128)], rowsb)
            for blk in range(NALLOC // 128):
                @pl.when(s == blk % 16)
                def _():
                    pltpu.sync_copy(rowsb, aggh.at[pl.ds(blk * 128, 128)])
            plsc.subcore_barrier()

            base = g * (EP // 128) + s * (EP // 128 // 16)

            def body(j, carry2):
                r = base + j
                pltpu.sync_copy(src2d.at[r], srcb)
                pltpu.sync_copy(dst2d.at[r], dstb)
                for k in range(8):
                    v = dstb[pl.ds(k * 16, 16)] - lo
                    ok = (v >= 0) & (v < NHALF)
                    dstl[pl.ds(k * 16, 16)] = jnp.where(
                        ok, v, jnp.int32(NHALF))
                pltpu.sync_copy(eawe.at[pl.ds(r * 128, 128)], eweb)
                pltpu.async_copy(hpre.at[srcb], rowsb, sem).wait()
                pltpu.sync_copy(rowsb, aggh.at[dstl], add=True)
                pltpu.sync_copy(eweb, aggh.at[dstl], add=True)
                return carry2

            lax.fori_loop(0, EP // 128 // 16, body, 0)
            plsc.subcore_barrier()

            out0 = g * NPAD + c * NHALF
            for blk in range(NHALF // 128):
                @pl.when(s == blk % 16)
                def _():
                    pltpu.sync_copy(aggh.at[pl.ds(blk * 128, 128)], rowsb)
                    pltpu.sync_copy(
                        rowsb, outh.at[pl.ds(out0 + blk * 128, 128)])
            plsc.subcore_barrier()
            return carry

        lax.fori_loop(0, 2, graph_pass, 0)

    return seg


_segsum = _make_segsum()


def _combine_body(hpre_ref, a_ref, b_ref, o_ref):
    o_ref[...] = jnp.maximum(hpre_ref[...] + a_ref[...] + b_ref[...], 0.0)


def _combine(hpre, aggh, b2d):
    bm = 1000
    return pl.pallas_call(
        _combine_body,
        grid=(N // bm,),
        in_specs=[pl.BlockSpec((bm, D), lambda i: (i, 0)),
                  pl.BlockSpec((bm, D), lambda i: (i, 0)),
                  pl.BlockSpec((1, D), lambda i: (0, 0))],
        out_specs=pl.BlockSpec((bm, D), lambda i: (i, 0)),
        out_shape=jax.ShapeDtypeStruct((N, D), jnp.float32),
    )(hpre, aggh[:N], b2d)


def _score_body(hs_ref, ht_ref, p_ref, idx_ref):
    hs = hs_ref[0]
    ht = ht_ref[0]
    nt = (((1,), (1,)), ((), ()))
    s = lax.dot_general(hs, ht, nt, preferred_element_type=jnp.float32)
    mink = 2.0 * hs[:, 0:1] * ht[:, 0][None, :]
    s = s - mink
    hs_hi = hs.astype(jnp.bfloat16).astype(jnp.float32)
    ht_hi = ht.astype(jnp.bfloat16).astype(jnp.float32)
    hs_lo = hs - hs_hi
    ht_lo = ht - ht_hi
    sv = lax.dot_general(hs_hi.astype(jnp.bfloat16),
                         ht_hi.astype(jnp.bfloat16), nt,
                         preferred_element_type=jnp.float32)
    sv = sv + lax.dot_general(hs_hi, ht_lo, nt,
                              preferred_element_type=jnp.float32)
    sv = sv + lax.dot_general(hs_lo, ht_hi, nt,
                              preferred_element_type=jnp.float32)
    sv = sv + lax.dot_general(hs_lo, ht_lo, nt,
                              preferred_element_type=jnp.float32)
    sv = sv - mink

    col = lax.broadcasted_iota(jnp.int32, (128, NSP), 1)
    neginf = jnp.float32(-jnp.inf)
    s = jnp.where(col < NS, s, neginf)

    vals = []
    idxs = []
    for _ in range(K):
        m = jnp.max(s, axis=1, keepdims=True)
        hit = s == m
        j = jnp.min(jnp.where(hit, col, jnp.int32(1 << 30)), axis=1,
                    keepdims=True)
        vals.append(jnp.sum(jnp.where(col == j, sv, 0.0), axis=1,
                            keepdims=True))
        idxs.append(j)
        s = jnp.where(col == j, neginf, s)

    v = jnp.concatenate(vals, axis=1)
    ix = jnp.concatenate(idxs, axis=1)
    e = jnp.exp(v - jnp.max(v, axis=1, keepdims=True))
    p = e / jnp.sum(e, axis=1, keepdims=True)
    p_ref[0] = p
    idx_ref[0] = ix


def _score_topk(hs_pad, ht_pad):
    return pl.pallas_call(
        _score_body,
        grid=(B, NSP // 128),
        in_specs=[pl.BlockSpec((1, 128, D), lambda b, i: (b, i, 0)),
                  pl.BlockSpec((1, NSP, D), lambda b, i: (b, 0, 0))],
        out_specs=(pl.BlockSpec((1, 128, K), lambda b, i: (b, i, 0)),
                   pl.BlockSpec((1, 128, K), lambda b, i: (b, i, 0))),
        out_shape=(jax.ShapeDtypeStruct((B, NSP, K), jnp.float32),
                   jax.ShapeDtypeStruct((B, NSP, K), jnp.int32)),
    )(hs_pad, ht_pad)


def _psi_half(x, w):
    return _matmul(x, w)


def kernel(x_s, edge_index_s, edge_attr_s, batch_s,
           x_t, edge_index_t, edge_attr_t, batch_t, W, We, b):
    xs2 = jnp.concatenate([x_s, x_t], axis=0)
    hpre = _matmul(xs2, W)
    hpre_s = hpre[:N]
    hpre_t = hpre[N:]

    zh = jnp.zeros((128, D), jnp.float32)

    def prep_edges(edge_index, edge_attr, src_off):
        src = jnp.concatenate(
            [edge_index[0] + src_off,
             jnp.full((EP - E,), src_off, jnp.int32)]).reshape(-1, 128)
        dst = jnp.concatenate(
            [edge_index[1],
             jnp.full((EP - E,), N, jnp.int32)]).reshape(-1, 128)
        ea = jnp.concatenate(
            [edge_attr, jnp.zeros((EP - E, DE), jnp.float32)], axis=0)
        return src, dst, ea

    src_s, dst_s, ea_s = prep_edges(edge_index_s, edge_attr_s, 0)
    src_t, dst_t, ea_t = prep_edges(edge_index_t, edge_attr_t, N)
    src2d = jnp.concatenate([src_s, src_t], axis=0)
    dst2d = jnp.concatenate([dst_s, dst_t], axis=0)
    ea2 = jnp.concatenate([ea_s, ea_t], axis=0)
    eawe = _matmul(ea2, We, bm=1024)

    aggh_all = _segsum(hpre, src2d, dst2d, eawe, zh)
    aggh_s = aggh_all[:NPAD]
    aggh_t = aggh_all[NPAD:]

    b2d = b.reshape(1, D)
    h_s = _combine(hpre_s, aggh_s, b2d)
    h_t = _combine(hpre_t, aggh_t, b2d)

    hs = h_s.reshape(B, NS, D)
    ht = h_t.reshape(B, NS, D)
    pad = ((0, 0), (0, NSP - NS), (0, 0))
    hs_pad = jnp.pad(hs, pad)
    ht_pad = jnp.pad(ht, pad)

    p, ix = _score_topk(hs_pad, ht_pad)
    S_0 = p[:, :NS].reshape(N, K)
    S_idx = ix[:, :NS].reshape(N, K)
    return S_0, S_idx


# speedup vs baseline: 2.5904x; 1.0703x over previous
import functools

import jax
import jax.numpy as jnp
from jax import lax
from jax.experimental import pallas as pl
from jax.experimental.pallas import tpu as pltpu
from jax.experimental.pallas import tpu_sc as plsc

N = 10000
B = 8
NS = N // B
E = 320000
D = 128
DE = 16
K = 32

NPAD = 10240
NSP = 1280
EP = 327680
NHALF = NPAD // 2
NALLOC = 5248


def _matmul_body(x_ref, w_ref, o_ref):
    o_ref[...] = jnp.dot(x_ref[...], w_ref[...],
                         preferred_element_type=jnp.float32)


def _matmul(x, w, bm=1000):
    m, k = x.shape
    return pl.pallas_call(
        _matmul_body,
        grid=(m // bm,),
        in_specs=[pl.BlockSpec((bm, k), lambda i: (i, 0)),
                  pl.BlockSpec((k, w.shape[1]), lambda i: (0, 0))],
        out_specs=pl.BlockSpec((bm, w.shape[1]), lambda i: (i, 0)),
        out_shape=jax.ShapeDtypeStruct((m, w.shape[1]), jnp.float32),
    )(x, w)


def _make_segsum():
    mesh = plsc.VectorSubcoreMesh(core_axis_name="c", subcore_axis_name="s")

    @functools.partial(
        pl.kernel,
        mesh=mesh,
        out_type=jax.ShapeDtypeStruct((2 * NPAD, D), jnp.float32),
        scratch_types=[
            pltpu.VMEM_SHARED((NALLOC, D), jnp.float32),
            pltpu.VMEM((2, 128), jnp.int32),
            pltpu.VMEM((2, 128), jnp.int32),
            pltpu.VMEM((128,), jnp.int32),
            pltpu.VMEM((128, D), jnp.float32),
            pltpu.VMEM((2, 128, D), jnp.float32),
            pltpu.SemaphoreType.DMA,
            pltpu.SemaphoreType.DMA,
            pltpu.SemaphoreType.DMA,
        ],
    )
    def seg(hpre, src2d, dst2d, eawe, zh, outh,
            aggh, srcb, dstb, dstl, rowsb, eweb, sem, isem0, isem1):
        c = lax.axis_index("c")
        s = lax.axis_index("s")
        lo = c * NHALF

        def graph_pass(g, carry):
            pltpu.sync_copy(zh.at[pl.ds(0, 128)], rowsb)
            for blk in range(NALLOC // 128):
                @pl.when(s == blk % 16)
                def _():
                    pltpu.sync_copy(rowsb, aggh.at[pl.ds(blk * 128, 128)])
            plsc.subcore_barrier()

            base = g * (EP // 128) + s * (EP // 128 // 16)
            nch = EP // 128 // 16

            def issue(jj, p):
                r = base + jj
                pltpu.async_copy(src2d.at[r], srcb.at[p], isem0)
                pltpu.async_copy(dst2d.at[r], dstb.at[p], isem0)
                pltpu.async_copy(eawe.at[pl.ds(r * 128, 128)],
                                 eweb.at[p], isem0)

            def drain(jj, p):
                r = base + jj
                pltpu.make_async_copy(src2d.at[r], srcb.at[p], isem0).wait()
                pltpu.make_async_copy(dst2d.at[r], dstb.at[p], isem0).wait()
                pltpu.make_async_copy(eawe.at[pl.ds(r * 128, 128)],
                                      eweb.at[p], isem0).wait()

            issue(0, 0)

            def body(j, carry2):
                p = j % 2
                drain(j, p)
                for k in range(8):
                    v = dstb[p, pl.ds(k * 16, 16)] - lo
                    ok = (v >= 0) & (v < NHALF)
                    dstl[pl.ds(k * 16, 16)] = jnp.where(
                        ok, v, jnp.int32(NHALF))
                gh = pltpu.async_copy(hpre.at[srcb.at[p]], rowsb, sem)

                @pl.when(j < nch - 1)
                def _():
                    issue(j + 1, 1 - p)

                gh.wait()
                pltpu.sync_copy(rowsb, aggh.at[dstl], add=True)
                pltpu.sync_copy(eweb.at[p], aggh.at[dstl], add=True)
                return carry2

            lax.fori_loop(0, nch, body, 0)
            plsc.subcore_barrier()

            out0 = g * NPAD + c * NHALF
            for blk in range(NHALF // 128):
                @pl.when(s == blk % 16)
                def _():
                    pltpu.sync_copy(aggh.at[pl.ds(blk * 128, 128)], rowsb)
                    pltpu.sync_copy(
                        rowsb, outh.at[pl.ds(out0 + blk * 128, 128)])
            plsc.subcore_barrier()
            return carry

        lax.fori_loop(0, 2, graph_pass, 0)

    return seg


_segsum = _make_segsum()


def _combine_body(hpre_ref, a_ref, b_ref, o_ref):
    o_ref[...] = jnp.maximum(hpre_ref[...] + a_ref[...] + b_ref[...], 0.0)


def _combine(hpre, aggh, b2d):
    bm = 1000
    return pl.pallas_call(
        _combine_body,
        grid=(N // bm,),
        in_specs=[pl.BlockSpec((bm, D), lambda i: (i, 0)),
                  pl.BlockSpec((bm, D), lambda i: (i, 0)),
                  pl.BlockSpec((1, D), lambda i: (0, 0))],
        out_specs=pl.BlockSpec((bm, D), lambda i: (i, 0)),
        out_shape=jax.ShapeDtypeStruct((N, D), jnp.float32),
    )(hpre, aggh[:N], b2d)


def _score_body(hs_ref, ht_ref, p_ref, idx_ref):
    hs = hs_ref[0]
    ht = ht_ref[0]
    nt = (((1,), (1,)), ((), ()))
    s = lax.dot_general(hs, ht, nt, preferred_element_type=jnp.float32)
    mink = 2.0 * hs[:, 0:1] * ht[:, 0][None, :]
    s = s - mink
    hs_hi = hs.astype(jnp.bfloat16).astype(jnp.float32)
    ht_hi = ht.astype(jnp.bfloat16).astype(jnp.float32)
    hs_lo = hs - hs_hi
    ht_lo = ht - ht_hi
    sv = lax.dot_general(hs_hi.astype(jnp.bfloat16),
                         ht_hi.astype(jnp.bfloat16), nt,
                         preferred_element_type=jnp.float32)
    sv = sv + lax.dot_general(hs_hi, ht_lo, nt,
                              preferred_element_type=jnp.float32)
    sv = sv + lax.dot_general(hs_lo, ht_hi, nt,
                              preferred_element_type=jnp.float32)
    sv = sv + lax.dot_general(hs_lo, ht_lo, nt,
                              preferred_element_type=jnp.float32)
    sv = sv - mink

    col = lax.broadcasted_iota(jnp.int32, (128, NSP), 1)
    neginf = jnp.float32(-jnp.inf)
    s = jnp.where(col < NS, s, neginf)

    vals = []
    idxs = []
    for _ in range(K):
        m = jnp.max(s, axis=1, keepdims=True)
        hit = s == m
        j = jnp.min(jnp.where(hit, col, jnp.int32(1 << 30)), axis=1,
                    keepdims=True)
        vals.append(jnp.sum(jnp.where(col == j, sv, 0.0), axis=1,
                            keepdims=True))
        idxs.append(j)
        s = jnp.where(col == j, neginf, s)

    v = jnp.concatenate(vals, axis=1)
    ix = jnp.concatenate(idxs, axis=1)
    e = jnp.exp(v - jnp.max(v, axis=1, keepdims=True))
    p = e / jnp.sum(e, axis=1, keepdims=True)
    p_ref[0] = p
    idx_ref[0] = ix


def _score_topk(hs_pad, ht_pad):
    return pl.pallas_call(
        _score_body,
        grid=(B, NSP // 128),
        in_specs=[pl.BlockSpec((1, 128, D), lambda b, i: (b, i, 0)),
                  pl.BlockSpec((1, NSP, D), lambda b, i: (b, 0, 0))],
        out_specs=(pl.BlockSpec((1, 128, K), lambda b, i: (b, i, 0)),
                   pl.BlockSpec((1, 128, K), lambda b, i: (b, i, 0))),
        out_shape=(jax.ShapeDtypeStruct((B, NSP, K), jnp.float32),
                   jax.ShapeDtypeStruct((B, NSP, K), jnp.int32)),
    )(hs_pad, ht_pad)


def _psi_half(x, w):
    return _matmul(x, w)


def kernel(x_s, edge_index_s, edge_attr_s, batch_s,
           x_t, edge_index_t, edge_attr_t, batch_t, W, We, b):
    xs2 = jnp.concatenate([x_s, x_t], axis=0)
    hpre = _matmul(xs2, W)
    hpre_s = hpre[:N]
    hpre_t = hpre[N:]

    zh = jnp.zeros((128, D), jnp.float32)

    def prep_edges(edge_index, edge_attr, src_off):
        src = jnp.concatenate(
            [edge_index[0] + src_off,
             jnp.full((EP - E,), src_off, jnp.int32)]).reshape(-1, 128)
        dst = jnp.concatenate(
            [edge_index[1],
             jnp.full((EP - E,), N, jnp.int32)]).reshape(-1, 128)
        ea = jnp.concatenate(
            [edge_attr, jnp.zeros((EP - E, DE), jnp.float32)], axis=0)
        return src, dst, ea

    src_s, dst_s, ea_s = prep_edges(edge_index_s, edge_attr_s, 0)
    src_t, dst_t, ea_t = prep_edges(edge_index_t, edge_attr_t, N)
    src2d = jnp.concatenate([src_s, src_t], axis=0)
    dst2d = jnp.concatenate([dst_s, dst_t], axis=0)
    ea2 = jnp.concatenate([ea_s, ea_t], axis=0)
    eawe = _matmul(ea2, We, bm=1024)

    aggh_all = _segsum(hpre, src2d, dst2d, eawe, zh)
    aggh_s = aggh_all[:NPAD]
    aggh_t = aggh_all[NPAD:]

    b2d = b.reshape(1, D)
    h_s = _combine(hpre_s, aggh_s, b2d)
    h_t = _combine(hpre_t, aggh_t, b2d)

    hs = h_s.reshape(B, NS, D)
    ht = h_t.reshape(B, NS, D)
    pad = ((0, 0), (0, NSP - NS), (0, 0))
    hs_pad = jnp.pad(hs, pad)
    ht_pad = jnp.pad(ht, pad)

    p, ix = _score_topk(hs_pad, ht_pad)
    S_0 = p[:, :NS].reshape(N, K)
    S_idx = ix[:, :NS].reshape(N, K)
    return S_0, S_idx


# speedup vs baseline: 2.6902x; 1.0385x over previous
import functools

import jax
import jax.numpy as jnp
from jax import lax
from jax.experimental import pallas as pl
from jax.experimental.pallas import tpu as pltpu
from jax.experimental.pallas import tpu_sc as plsc

N = 10000
B = 8
NS = N // B
E = 320000
D = 128
DE = 16
K = 32

NPAD = 10240
NSP = 1280
EP = 327680
NHALF = NPAD // 2
NALLOC = 5248


def _matmul_body(x_ref, w_ref, o_ref):
    o_ref[...] = jnp.dot(x_ref[...], w_ref[...],
                         preferred_element_type=jnp.float32)


def _matmul(x, w, bm=1000):
    m, k = x.shape
    return pl.pallas_call(
        _matmul_body,
        grid=(m // bm,),
        in_specs=[pl.BlockSpec((bm, k), lambda i: (i, 0)),
                  pl.BlockSpec((k, w.shape[1]), lambda i: (0, 0))],
        out_specs=pl.BlockSpec((bm, w.shape[1]), lambda i: (i, 0)),
        out_shape=jax.ShapeDtypeStruct((m, w.shape[1]), jnp.float32),
    )(x, w)


def _make_segsum():
    mesh = plsc.VectorSubcoreMesh(core_axis_name="c", subcore_axis_name="s")

    @functools.partial(
        pl.kernel,
        mesh=mesh,
        out_type=jax.ShapeDtypeStruct((2 * NPAD, D), jnp.float32),
        scratch_types=[
            pltpu.VMEM_SHARED((NALLOC, D), jnp.float32),
            pltpu.VMEM((2, 128), jnp.int32),
            pltpu.VMEM((2, 128), jnp.int32),
            pltpu.VMEM((2, 128), jnp.int32),
            pltpu.VMEM((2, 128, D), jnp.float32),
            pltpu.VMEM((2, 128, D), jnp.float32),
            pltpu.SemaphoreType.DMA,
            pltpu.SemaphoreType.DMA,
            pltpu.SemaphoreType.DMA,
        ],
    )
    def seg(hpre, src2d, dst2d, eawe, zh, outh,
            aggh, srcb, dstb, dstl, rowsb, eweb, sem, isem0, ssem):
        c = lax.axis_index("c")
        s = lax.axis_index("s")
        lo = c * NHALF

        def graph_pass(g, carry):
            pltpu.sync_copy(zh.at[pl.ds(0, 128)], rowsb.at[0])
            for blk in range(NALLOC // 128):
                @pl.when(s == blk % 16)
                def _():
                    pltpu.sync_copy(rowsb.at[0],
                                    aggh.at[pl.ds(blk * 128, 128)])
            plsc.subcore_barrier()

            base = g * (EP // 128) + s * (EP // 128 // 16)
            nch = EP // 128 // 16

            def issue(jj, p):
                r = base + jj
                pltpu.async_copy(src2d.at[r], srcb.at[p], isem0)
                pltpu.async_copy(dst2d.at[r], dstb.at[p], isem0)
                pltpu.async_copy(eawe.at[pl.ds(r * 128, 128)],
                                 eweb.at[p], isem0)

            def drain(jj, p):
                r = base + jj
                pltpu.make_async_copy(src2d.at[r], srcb.at[p], isem0).wait()
                pltpu.make_async_copy(dst2d.at[r], dstb.at[p], isem0).wait()
                pltpu.make_async_copy(eawe.at[pl.ds(r * 128, 128)],
                                      eweb.at[p], isem0).wait()

            def drain_scatter(p):
                pltpu.make_async_copy(
                    rowsb.at[p], aggh.at[dstl.at[p]], ssem).wait()
                pltpu.make_async_copy(
                    eweb.at[p], aggh.at[dstl.at[p]], ssem).wait()

            issue(0, 0)

            def body(j, carry2):
                p = j % 2
                drain(j, p)
                for k in range(8):
                    v = dstb[p, pl.ds(k * 16, 16)] - lo
                    ok = (v >= 0) & (v < NHALF)
                    dstl[p, pl.ds(k * 16, 16)] = jnp.where(
                        ok, v, jnp.int32(NHALF))
                gh = pltpu.async_copy(hpre.at[srcb.at[p]], rowsb.at[p], sem)

                @pl.when(j >= 1)
                def _():
                    drain_scatter(1 - p)

                @pl.when(j < nch - 1)
                def _():
                    issue(j + 1, 1 - p)

                gh.wait()
                pltpu.async_copy(rowsb.at[p], aggh.at[dstl.at[p]], ssem,
                                 add=True)
                pltpu.async_copy(eweb.at[p], aggh.at[dstl.at[p]], ssem,
                                 add=True)
                return carry2

            lax.fori_loop(0, nch, body, 0)
            drain_scatter((nch - 1) % 2)
            plsc.subcore_barrier()

            out0 = g * NPAD + c * NHALF
            for blk in range(NHALF // 128):
                @pl.when(s == blk % 16)
                def _():
                    pltpu.sync_copy(aggh.at[pl.ds(blk * 128, 128)],
                                    rowsb.at[0])
                    pltpu.sync_copy(
                        rowsb.at[0], outh.at[pl.ds(out0 + blk * 128, 128)])
            plsc.subcore_barrier()
            return carry

        lax.fori_loop(0, 2, graph_pass, 0)

    return seg


_segsum = _make_segsum()


def _combine_body(hpre_ref, a_ref, b_ref, o_ref):
    o_ref[...] = jnp.maximum(hpre_ref[...] + a_ref[...] + b_ref[...], 0.0)


def _combine(hpre, aggh, b2d):
    bm = 1000
    return pl.pallas_call(
        _combine_body,
        grid=(N // bm,),
        in_specs=[pl.BlockSpec((bm, D), lambda i: (i, 0)),
                  pl.BlockSpec((bm, D), lambda i: (i, 0)),
                  pl.BlockSpec((1, D), lambda i: (0, 0))],
        out_specs=pl.BlockSpec((bm, D), lambda i: (i, 0)),
        out_shape=jax.ShapeDtypeStruct((N, D), jnp.float32),
    )(hpre, aggh[:N], b2d)


def _score_body(hs_ref, ht_ref, p_ref, idx_ref):
    hs = hs_ref[0]
    ht = ht_ref[0]
    nt = (((1,), (1,)), ((), ()))
    s = lax.dot_general(hs, ht, nt, preferred_element_type=jnp.float32)
    mink = 2.0 * hs[:, 0:1] * ht[:, 0][None, :]
    s = s - mink
    hs_hi = hs.astype(jnp.bfloat16).astype(jnp.float32)
    ht_hi = ht.astype(jnp.bfloat16).astype(jnp.float32)
    hs_lo = hs - hs_hi
    ht_lo = ht - ht_hi
    sv = lax.dot_general(hs_hi.astype(jnp.bfloat16),
                         ht_hi.astype(jnp.bfloat16), nt,
                         preferred_element_type=jnp.float32)
    sv = sv + lax.dot_general(hs_hi, ht_lo, nt,
                              preferred_element_type=jnp.float32)
    sv = sv + lax.dot_general(hs_lo, ht_hi, nt,
                              preferred_element_type=jnp.float32)
    sv = sv + lax.dot_general(hs_lo, ht_lo, nt,
                              preferred_element_type=jnp.float32)
    sv = sv - mink

    col = lax.broadcasted_iota(jnp.int32, (128, NSP), 1)
    neginf = jnp.float32(-jnp.inf)
    s = jnp.where(col < NS, s, neginf)

    vals = []
    idxs = []
    for _ in range(K):
        m = jnp.max(s, axis=1, keepdims=True)
        hit = s == m
        j = jnp.min(jnp.where(hit, col, jnp.int32(1 << 30)), axis=1,
                    keepdims=True)
        vals.append(jnp.sum(jnp.where(col == j, sv, 0.0), axis=1,
                            keepdims=True))
        idxs.append(j)
        s = jnp.where(col == j, neginf, s)

    v = jnp.concatenate(vals, axis=1)
    ix = jnp.concatenate(idxs, axis=1)
    e = jnp.exp(v - jnp.max(v, axis=1, keepdims=True))
    p = e / jnp.sum(e, axis=1, keepdims=True)
    p_ref[0] = p
    idx_ref[0] = ix


def _score_topk(hs_pad, ht_pad):
    return pl.pallas_call(
        _score_body,
        grid=(B, NSP // 128),
        in_specs=[pl.BlockSpec((1, 128, D), lambda b, i: (b, i, 0)),
                  pl.BlockSpec((1, NSP, D), lambda b, i: (b, 0, 0))],
        out_specs=(pl.BlockSpec((1, 128, K), lambda b, i: (b, i, 0)),
                   pl.BlockSpec((1, 128, K), lambda b, i: (b, i, 0))),
        out_shape=(jax.ShapeDtypeStruct((B, NSP, K), jnp.float32),
                   jax.ShapeDtypeStruct((B, NSP, K), jnp.int32)),
    )(hs_pad, ht_pad)


def _psi_half(x, w):
    return _matmul(x, w)


def kernel(x_s, edge_index_s, edge_attr_s, batch_s,
           x_t, edge_index_t, edge_attr_t, batch_t, W, We, b):
    xs2 = jnp.concatenate([x_s, x_t], axis=0)
    hpre = _matmul(xs2, W)
    hpre_s = hpre[:N]
    hpre_t = hpre[N:]

    zh = jnp.zeros((128, D), jnp.float32)

    def prep_edges(edge_index, edge_attr, src_off):
        src = jnp.concatenate(
            [edge_index[0] + src_off,
             jnp.full((EP - E,), src_off, jnp.int32)]).reshape(-1, 128)
        dst = jnp.concatenate(
            [edge_index[1],
             jnp.full((EP - E,), N, jnp.int32)]).reshape(-1, 128)
        ea = jnp.concatenate(
            [edge_attr, jnp.zeros((EP - E, DE), jnp.float32)], axis=0)
        return src, dst, ea

    src_s, dst_s, ea_s = prep_edges(edge_index_s, edge_attr_s, 0)
    src_t, dst_t, ea_t = prep_edges(edge_index_t, edge_attr_t, N)
    src2d = jnp.concatenate([src_s, src_t], axis=0)
    dst2d = jnp.concatenate([dst_s, dst_t], axis=0)
    ea2 = jnp.concatenate([ea_s, ea_t], axis=0)
    eawe = _matmul(ea2, We, bm=1024)

    aggh_all = _segsum(hpre, src2d, dst2d, eawe, zh)
    aggh_s = aggh_all[:NPAD]
    aggh_t = aggh_all[NPAD:]

    b2d = b.reshape(1, D)
    h_s = _combine(hpre_s, aggh_s, b2d)
    h_t = _combine(hpre_t, aggh_t, b2d)

    hs = h_s.reshape(B, NS, D)
    ht = h_t.reshape(B, NS, D)
    pad = ((0, 0), (0, NSP - NS), (0, 0))
    hs_pad = jnp.pad(hs, pad)
    ht_pad = jnp.pad(ht, pad)

    p, ix = _score_topk(hs_pad, ht_pad)
    S_0 = p[:, :NS].reshape(N, K)
    S_idx = ix[:, :NS].reshape(N, K)
    return S_0, S_idx
